# Initial kernel scaffold; baseline (speedup 1.0000x reference)
import jax
import jax.numpy as jnp
from jax.experimental import pallas as pl


def kernel(x, cnn_conv_w, cnn_conv_b, cnn_lin_w, cnn_lin_b, gat_lin_w, att_src, att_dst, gat_bias, out_lin_w, out_lin_b, edge_index):
    raise NotImplementedError("write your pallas kernel here")


# speedup vs baseline: 14.3408x; 14.3408x over previous
import functools

import jax
import jax.numpy as jnp
import numpy as np
from jax import lax
from jax.experimental import pallas as pl
from jax.experimental.pallas import tpu as pltpu
import jax.experimental.pallas.tpu_sc as plsc

N = 50000
E = 800000
HID = 20
CNN_CH = 16
KERN = 8
STRIDE = 4
NPOS = 97
PPAD = 128
FIN = 392
BN = 1000
K = 128
NW = 32
NCHUNKS = E // K
RA = 3128
RB = N - 15 * RA


def _dense_body(x_ref, wb_ref, bmat_ref, m1_ref, c1g_ref, attw_ref,
                o1_ref, o2_ref):
    xb = x_ref[:, 1, :]
    avg = xb[:, 0:1]
    f = xb[:, 3:395]
    hf = jnp.dot(f, wb_ref[...], preferred_element_type=jnp.float32)
    hf = hf.reshape(BN, CNN_CH, PPAD) + bmat_ref[...][None, :, :]
    m16 = jnp.maximum(hf, 0.0).sum(axis=2) * (1.0 / NPOS)
    hx = (jnp.dot(m16, m1_ref[...], preferred_element_type=jnp.float32)
          + avg * c1g_ref[0:1, :] + c1g_ref[1:2, :])
    a12 = jnp.dot(hx, attw_ref[...], preferred_element_type=jnp.float32)
    o1_ref[:, 0:20] = hx
    o1_ref[:, 20:21] = a12[:, 0:1]
    o1_ref[:, 21:32] = jnp.zeros((BN, 11), jnp.float32)
    o2_ref[:, 0:1] = a12[:, 1:2]
    o2_ref[:, 1:16] = jnp.zeros((BN, 15), jnp.float32)


def _edge_body(ei_hbm, hx_hbm, ad_hbm, z_hbm, out_hbm,
               srcv, dstv, gv, av, mv, s_acc, sem1, sem2):
    cid = lax.axis_index("c")
    sid = lax.axis_index("s")
    wid = sid * 2 + cid

    @pl.when(sid < 15)
    def _():
        pltpu.sync_copy(z_hbm.at[pl.ds(sid * RA, RA)],
                        s_acc.at[pl.ds(sid * RA, RA)])

    @pl.when(sid == 15)
    def _():
        pltpu.sync_copy(z_hbm.at[pl.ds(15 * RA, RB)],
                        s_acc.at[pl.ds(15 * RA, RB)])

    plsc.subcore_barrier()

    nch = NCHUNKS // NW + jnp.where(wid < NCHUNKS % NW, 1, 0)
    lane = lax.iota(jnp.int32, 16)
    zv16 = jnp.zeros((16,), jnp.float32)

    def zrow(jj, c2):
        mv[jj, pl.ds(16, 16)] = zv16
        return c2

    lax.fori_loop(0, K, zrow, 0)

    def chunk(j, carry):
        off = (wid + NW * j) * K
        pltpu.sync_copy(ei_hbm.at[pl.ds(off, K)], srcv)
        pltpu.sync_copy(ei_hbm.at[pl.ds(E + off, K)], dstv)
        d1 = pltpu.async_copy(hx_hbm.at[srcv], gv, sem1)
        d2 = pltpu.async_copy(ad_hbm.at[dstv], av, sem2)
        d1.wait()
        d2.wait()
        for g in range(K // 16):
            rows = lane + (g * 16)
            asrc = plsc.load_gather(gv, [rows, jnp.full((16,), 20, jnp.int32)])
            ad = plsc.load_gather(av, [rows, jnp.zeros((16,), jnp.int32)])
            t = asrc + ad
            e = jnp.where(t >= 0.0, t, t * 0.2)
            w = jnp.exp(e)
            plsc.store_scatter(mv, [rows, jnp.full((16,), 20, jnp.int32)], w)
            for c in range(HID):
                ci = jnp.full((16,), c, jnp.int32)
                h = plsc.load_gather(gv, [rows, ci])
                plsc.store_scatter(mv, [rows, ci], h * w)

        pltpu.sync_copy(mv, s_acc.at[dstv], add=True)
        return carry

    lax.fori_loop(0, nch, chunk, 0)
    plsc.subcore_barrier()

    @pl.when(sid < 15)
    def _():
        pltpu.sync_copy(s_acc.at[pl.ds(sid * RA, RA)],
                        out_hbm.at[cid, pl.ds(sid * RA, RA)])

    @pl.when(sid == 15)
    def _():
        pltpu.sync_copy(s_acc.at[pl.ds(15 * RA, RB)],
                        out_hbm.at[cid, pl.ds(15 * RA, RB)])


def _finish_body(p_ref, ow_ref, ob_ref, y_ref):
    s = p_ref[0] + p_ref[1]
    num = s[:, 0:20]
    den = s[:, 20:21]
    agg = num / (den + 1e-16) + ow_ref[0:1, :]
    h1 = jnp.maximum(agg, 0.0)
    y_ref[...] = (jnp.sum(h1 * ow_ref[1:2, :], axis=1, keepdims=True)
                  + ob_ref[0, 0])


def kernel(x, cnn_conv_w, cnn_conv_b, cnn_lin_w, cnn_lin_b,
           gat_lin_w, att_src, att_dst, gat_bias,
           out_lin_w, out_lin_b, edge_index):
    f32 = jnp.float32
    cs = np.arange(CNN_CH)[:, None, None]
    ps = np.arange(NPOS)[None, :, None]
    ks = np.arange(KERN)[None, None, :]
    qidx = np.broadcast_to(STRIDE * ps + ks, (CNN_CH, NPOS, KERN)).ravel()
    cols = np.broadcast_to(cs * PPAD + ps, (CNN_CH, NPOS, KERN)).ravel()
    vals = jnp.broadcast_to(cnn_conv_w[:, 0, :][:, None, :],
                            (CNN_CH, NPOS, KERN)).reshape(-1)
    wb = jnp.zeros((FIN, CNN_CH * PPAD), f32).at[qidx, cols].set(vals)
    bmat = (cnn_conv_b[:, None]
            * (np.arange(PPAD) < NPOS)[None, :].astype(np.float32))
    g80t = gat_lin_w[:, :80].T
    m1 = cnn_lin_w.T @ g80t
    c1 = cnn_lin_b @ g80t
    c1g = jnp.stack([gat_lin_w[:, 80], c1])
    attw = jnp.stack([att_src, att_dst], axis=1)
    ow = jnp.stack([gat_bias, out_lin_w[0]])
    ob = out_lin_b.reshape(1, 1)

    nblk = N // BN
    hx32, ad16 = pl.pallas_call(
        _dense_body,
        grid=(nblk,),
        in_specs=[
            pl.BlockSpec((BN, 2, 395), lambda i: (i, 0, 0)),
            pl.BlockSpec((FIN, CNN_CH * PPAD), lambda i: (0, 0)),
            pl.BlockSpec((CNN_CH, PPAD), lambda i: (0, 0)),
            pl.BlockSpec((CNN_CH, HID), lambda i: (0, 0)),
            pl.BlockSpec((2, HID), lambda i: (0, 0)),
            pl.BlockSpec((HID, 2), lambda i: (0, 0)),
        ],
        out_specs=[
            pl.BlockSpec((BN, 32), lambda i: (i, 0)),
            pl.BlockSpec((BN, 16), lambda i: (i, 0)),
        ],
        out_shape=[
            jax.ShapeDtypeStruct((N, 32), f32),
            jax.ShapeDtypeStruct((N, 16), f32),
        ],
    )(x, wb, bmat, m1, c1g, attw)

    mesh = plsc.VectorSubcoreMesh(core_axis_name="c", subcore_axis_name="s",
                                  num_cores=2, num_subcores=16)
    zeros = jnp.zeros((N, 32), f32)
    edge_call = functools.partial(
        pl.kernel,
        out_type=jax.ShapeDtypeStruct((2, N, 32), f32),
        mesh=mesh,
        compiler_params=pltpu.CompilerParams(needs_layout_passes=False,
                                             use_tc_tiling_on_sc=False),
        scratch_types=[
            pltpu.VMEM((K,), jnp.int32),
            pltpu.VMEM((K,), jnp.int32),
            pltpu.VMEM((K, 32), f32),
            pltpu.VMEM((K, 16), f32),
            pltpu.VMEM((K, 32), f32),
            pltpu.VMEM_SHARED((N, 32), f32),
            pltpu.SemaphoreType.DMA,
            pltpu.SemaphoreType.DMA,
        ],
    )(_edge_body)
    partials = edge_call(edge_index.reshape(-1), hx32, ad16, zeros)

    nblk2 = N // BN
    y = pl.pallas_call(
        _finish_body,
        grid=(nblk2,),
        in_specs=[
            pl.BlockSpec((2, BN, 32), lambda i: (0, i, 0)),
            pl.BlockSpec((2, HID), lambda i: (0, 0)),
            pl.BlockSpec(memory_space=pltpu.SMEM),
        ],
        out_specs=pl.BlockSpec((BN, 1), lambda i: (i, 0)),
        out_shape=jax.ShapeDtypeStruct((N, 1), f32),
    )(partials, ow, ob)
    return y


# speedup vs baseline: 22.9958x; 1.6035x over previous
import functools

import jax
import jax.numpy as jnp
import numpy as np
from jax import lax
from jax.experimental import pallas as pl
from jax.experimental.pallas import tpu as pltpu
import jax.experimental.pallas.tpu_sc as plsc

N = 50000
E = 800000
HID = 20
CNN_CH = 16
KERN = 8
STRIDE = 4
NPOS = 97
PPAD = 128
FIN = 392
BN = 1000
K = 128
NW = 32
EPW = E // NW
NCH = 196
TAIL = EPW - 195 * K
D = 24
RA = 3128
RB = N - 15 * RA


def _dense_body(x_ref, wb_ref, bvec_ref, obd_ref, m1_ref, c1g_ref, attw_ref,
                o1_ref, o2_ref):
    xb = x_ref[:, 1, :]
    avg = xb[:, 0:1]
    f = xb.astype(jnp.bfloat16)
    hf = jnp.dot(f, wb_ref[...], preferred_element_type=jnp.float32)
    hr = jnp.maximum(hf + bvec_ref[0:1, :], 0.0)
    m16 = jnp.dot(hr, obd_ref[...],
                  preferred_element_type=jnp.float32) * (1.0 / NPOS)
    hx = (jnp.dot(m16, m1_ref[...], preferred_element_type=jnp.float32)
          + avg * c1g_ref[0:1, :] + c1g_ref[1:2, :])
    a12 = jnp.dot(hx, attw_ref[...], preferred_element_type=jnp.float32)
    o1_ref[:, 0:20] = hx
    o1_ref[:, 20:21] = a12[:, 0:1]
    o1_ref[:, 21:24] = jnp.zeros((BN, 3), jnp.float32)
    o2_ref[:, 0:1] = a12[:, 1:2]
    o2_ref[:, 1:16] = jnp.zeros((BN, 15), jnp.float32)


def _edge_body(em_hbm, hx_hbm, ad_hbm, out_hbm,
               sloc, dloc, gv0, av0, mv0, gv1, av1, mv1,
               s_acc, gsem0, gsem1, ssem0, ssem1):
    cid = lax.axis_index("c")
    sid = lax.axis_index("s")
    wid = sid * 2 + cid

    lane = lax.iota(jnp.int32, 16)
    z16i = jnp.zeros((16,), jnp.int32)
    zf = jnp.zeros((16,), jnp.float32)
    c20 = jnp.full((16,), 20, jnp.int32)

    def zrow(jj, c2):
        for mv in (mv0, mv1):
            mv[jj, pl.ds(0, 16)] = zf
            mv[jj, pl.ds(8, 16)] = zf
        return c2

    lax.fori_loop(0, K, zrow, 0)

    base = sid * RA
    for t in range(24):
        pltpu.sync_copy(mv0, s_acc.at[pl.ds(base + t * K, K)])

    @pl.when(sid < 15)
    def _():
        pltpu.sync_copy(mv0.at[pl.ds(0, RA - 24 * K)],
                        s_acc.at[pl.ds(base + 24 * K, RA - 24 * K)])

    @pl.when(sid == 15)
    def _():
        pltpu.sync_copy(mv0.at[pl.ds(0, RB - 24 * K)],
                        s_acc.at[pl.ds(base + 24 * K, RB - 24 * K)])

    plsc.subcore_barrier()

    def g_descs(jl, gv, av, gsem):
        return (pltpu.make_async_copy(hx_hbm.at[sloc.at[jl]], gv, gsem),
                pltpu.make_async_copy(ad_hbm.at[dloc.at[jl]], av, gsem))

    def start_gathers(jl, gv, av, gsem):
        d1, d2 = g_descs(jl, gv, av, gsem)
        d1.start()
        d2.start()

    def wait_gathers(jl, gv, av, gsem):
        d1, d2 = g_descs(jl, gv, av, gsem)
        d1.wait()
        d2.wait()

    def start_scatter(jl, mv, ssem):
        pltpu.make_async_copy(mv, s_acc.at[dloc.at[jl]], ssem).start(add=True)

    def wait_scatter(jl, mv, ssem):
        pltpu.make_async_copy(mv, s_acc.at[dloc.at[jl]], ssem).wait()

    def group(gv, av, mv, g, msk):
        rows = lane + (g * 16)
        asrc = plsc.load_gather(gv, [rows, c20])
        ad = plsc.load_gather(av, [rows, z16i])
        t = asrc + ad
        e = jnp.where(t >= 0.0, t, t * 0.2)
        w = jnp.exp(e)
        plsc.store_scatter(mv, [rows, c20], w, mask=msk)
        for c in range(HID):
            ci = jnp.full((16,), c, jnp.int32)
            h = plsc.load_gather(gv, [rows, ci])
            plsc.store_scatter(mv, [rows, ci], h * w, mask=msk)

    def compute_chunk(gv, av, mv):
        for g in range(K // 16):
            group(gv, av, mv, g, None)

    bufs = ((gv0, av0, mv0, gsem0, ssem0), (gv1, av1, mv1, gsem1, ssem1))

    HC = NCH // 2
    for half in range(2):
        pltpu.sync_copy(em_hbm.at[0, wid, pl.ds(HC * half, HC)], sloc)
        pltpu.sync_copy(em_hbm.at[1, wid, pl.ds(HC * half, HC)], dloc)

        start_gathers(0, gv0, av0, gsem0)
        start_gathers(1, gv1, av1, gsem1)

        def pair(i, carry):
            for b in range(2):
                gv, av, mv, gsem, ssem = bufs[b]
                jl = 2 * i + b
                wait_gathers(jl, gv, av, gsem)

                @pl.when(i > 0)
                def _():
                    wait_scatter(jl - 2, mv, ssem)

                compute_chunk(gv, av, mv)
                start_scatter(jl, mv, ssem)
                start_gathers(jl + 2, gv, av, gsem)
            return carry

        lax.fori_loop(0, HC // 2 - 1, pair, 0)

        wait_gathers(HC - 2, gv0, av0, gsem0)
        wait_scatter(HC - 4, mv0, ssem0)
        compute_chunk(gv0, av0, mv0)
        start_scatter(HC - 2, mv0, ssem0)

        wait_gathers(HC - 1, gv1, av1, gsem1)
        wait_scatter(HC - 3, mv1, ssem1)
        compute_chunk(gv1, av1, mv1)
        start_scatter(HC - 1, mv1, ssem1)

        wait_scatter(HC - 2, mv0, ssem0)
        wait_scatter(HC - 1, mv1, ssem1)

    plsc.subcore_barrier()

    @pl.when(sid < 15)
    def _():
        pltpu.sync_copy(s_acc.at[pl.ds(sid * RA, RA)],
                        out_hbm.at[cid, pl.ds(sid * RA, RA)])

    @pl.when(sid == 15)
    def _():
        pltpu.sync_copy(s_acc.at[pl.ds(15 * RA, RB)],
                        out_hbm.at[cid, pl.ds(15 * RA, RB)])


def _finish_body(p_ref, ow_ref, ob_ref, y_ref):
    s = p_ref[0] + p_ref[1]
    num = s[:, 0:20]
    den = s[:, 20:21]
    agg = num / (den + 1e-16) + ow_ref[0:1, :]
    h1 = jnp.maximum(agg, 0.0)
    y_ref[...] = (jnp.sum(h1 * ow_ref[1:2, :], axis=1, keepdims=True)
                  + ob_ref[0, 0])


def kernel(x, cnn_conv_w, cnn_conv_b, cnn_lin_w, cnn_lin_b,
           gat_lin_w, att_src, att_dst, gat_bias,
           out_lin_w, out_lin_b, edge_index):
    f32 = jnp.float32
    ind = np.zeros((395, PPAD, KERN), np.float32)
    for p in range(NPOS):
        for k in range(KERN):
            ind[3 + STRIDE * p + k, p, k] = 1.0
    w2 = cnn_conv_w[:, 0, :]
    wb = (ind.reshape(395 * PPAD, KERN) @ w2.T)
    wb = (wb.reshape(395, PPAD, CNN_CH).transpose(0, 2, 1)
          .reshape(395, CNN_CH * PPAD)).astype(jnp.bfloat16)
    pmask = (np.arange(PPAD) < NPOS).astype(np.float32)
    bvec = (cnn_conv_b[:, None] * pmask[None, :]).reshape(1, CNN_CH * PPAD)
    obd = np.zeros((CNN_CH * PPAD, CNN_CH), np.float32)
    for c in range(CNN_CH):
        obd[c * PPAD:c * PPAD + NPOS, c] = 1.0
    obd = jnp.asarray(obd)
    g80t = gat_lin_w[:, :80].T
    m1 = cnn_lin_w.T @ g80t
    c1 = cnn_lin_b @ g80t
    c1g = jnp.stack([gat_lin_w[:, 80], c1])
    attw = jnp.stack([att_src, att_dst], axis=1)
    ow = jnp.stack([gat_bias, out_lin_w[0]])
    ob = out_lin_b.reshape(1, 1)

    nblk = N // BN
    hx32, ad16 = pl.pallas_call(
        _dense_body,
        grid=(nblk,),
        in_specs=[
            pl.BlockSpec((BN, 2, 395), lambda i: (i, 0, 0)),
            pl.BlockSpec((395, CNN_CH * PPAD), lambda i: (0, 0)),
            pl.BlockSpec((1, CNN_CH * PPAD), lambda i: (0, 0)),
            pl.BlockSpec((CNN_CH * PPAD, CNN_CH), lambda i: (0, 0)),
            pl.BlockSpec((CNN_CH, HID), lambda i: (0, 0)),
            pl.BlockSpec((2, HID), lambda i: (0, 0)),
            pl.BlockSpec((HID, 2), lambda i: (0, 0)),
        ],
        out_specs=[
            pl.BlockSpec((BN, D), lambda i: (i, 0)),
            pl.BlockSpec((BN, 16), lambda i: (i, 0)),
        ],
        out_shape=[
            jax.ShapeDtypeStruct((N, D), f32),
            jax.ShapeDtypeStruct((N, 16), f32),
        ],
    )(x, wb, bvec, obd, m1, c1g, attw)

    mesh = plsc.VectorSubcoreMesh(core_axis_name="c", subcore_axis_name="s",
                                  num_cores=2, num_subcores=16)
    pad = NW * NCH * K - E
    srcp = jnp.pad(edge_index[0], (0, pad))
    dstp = jnp.pad(edge_index[1], (0, pad), constant_values=N)
    em = jnp.stack([srcp, dstp]).reshape(2, NW, NCH, K)
    adp = jnp.pad(ad16, ((0, 8), (0, 0)))
    edge_call = functools.partial(
        pl.kernel,
        out_type=jax.ShapeDtypeStruct((2, N, D), f32),
        mesh=mesh,
        compiler_params=pltpu.CompilerParams(needs_layout_passes=False,
                                             use_tc_tiling_on_sc=False),
        scratch_types=[
            pltpu.VMEM((NCH // 2, K), jnp.int32),
            pltpu.VMEM((NCH // 2, K), jnp.int32),
            pltpu.VMEM((K, D), f32),
            pltpu.VMEM((K, 16), f32),
            pltpu.VMEM((K, D), f32),
            pltpu.VMEM((K, D), f32),
            pltpu.VMEM((K, 16), f32),
            pltpu.VMEM((K, D), f32),
            pltpu.VMEM_SHARED((N + 8, D), f32),
            pltpu.SemaphoreType.DMA,
            pltpu.SemaphoreType.DMA,
            pltpu.SemaphoreType.DMA,
            pltpu.SemaphoreType.DMA,
        ],
    )(_edge_body)
    partials = edge_call(em, hx32, adp)

    nblk2 = N // BN
    y = pl.pallas_call(
        _finish_body,
        grid=(nblk2,),
        in_specs=[
            pl.BlockSpec((2, BN, D), lambda i: (0, i, 0)),
            pl.BlockSpec((2, HID), lambda i: (0, 0)),
            pl.BlockSpec(memory_space=pltpu.SMEM),
        ],
        out_specs=pl.BlockSpec((BN, 1), lambda i: (i, 0)),
        out_shape=jax.ShapeDtypeStruct((N, 1), f32),
    )(partials, ow, ob)
    return y


# speedup vs baseline: 25.9789x; 1.1297x over previous
import functools

import jax
import jax.numpy as jnp
import numpy as np
from jax import lax
from jax.experimental import pallas as pl
from jax.experimental.pallas import tpu as pltpu
import jax.experimental.pallas.tpu_sc as plsc

N = 50000
E = 800000
HID = 20
CNN_CH = 16
KERN = 8
STRIDE = 4
NPOS = 97
PPAD = 128
FIN = 392
BN = 1000
K = 128
NW = 32
EPW = E // NW
NCH = 196
TAIL = EPW - 195 * K
D = 24
RA = 3128
RB = N - 15 * RA


def _dense_body(x_ref, wb_ref, bvec_ref, obd_ref, m1_ref, c1g_ref, attw_ref,
                o1_ref, o2_ref):
    xb = x_ref[...]
    avg = xb[:, 0:1]
    f = xb.astype(jnp.bfloat16)
    hf = jnp.dot(f, wb_ref[...], preferred_element_type=jnp.float32)
    hr = jnp.maximum(hf + bvec_ref[0:1, :], 0.0)
    m16 = jnp.dot(hr, obd_ref[...],
                  preferred_element_type=jnp.float32) * (1.0 / NPOS)
    hx = (jnp.dot(m16, m1_ref[...], preferred_element_type=jnp.float32)
          + avg * c1g_ref[0:1, :] + c1g_ref[1:2, :])
    a12 = jnp.dot(hx, attw_ref[...], preferred_element_type=jnp.float32)
    o1_ref[:, 0:20] = hx
    o1_ref[:, 20:21] = a12[:, 0:1]
    o1_ref[:, 21:24] = jnp.zeros((BN, 3), jnp.float32)
    o2_ref[:, 0:1] = a12[:, 1:2]
    o2_ref[:, 1:16] = jnp.zeros((BN, 15), jnp.float32)


def _edge_body(em_hbm, hx_hbm, ad_hbm, out_hbm,
               sloc, dloc, gv0, av0, mv0, gv1, av1, mv1,
               s_acc, gsem0, gsem1, ssem0, ssem1):
    cid = lax.axis_index("c")
    sid = lax.axis_index("s")
    wid = sid * 2 + cid

    lane = lax.iota(jnp.int32, 16)
    z16i = jnp.zeros((16,), jnp.int32)
    zf = jnp.zeros((16,), jnp.float32)
    c20 = jnp.full((16,), 20, jnp.int32)

    def zrow(jj, c2):
        for mv in (mv0, mv1):
            mv[jj, pl.ds(0, 16)] = zf
            mv[jj, pl.ds(8, 16)] = zf
        return c2

    lax.fori_loop(0, K, zrow, 0)

    base = sid * RA
    for t in range(24):
        pltpu.sync_copy(mv0, s_acc.at[pl.ds(base + t * K, K)])

    @pl.when(sid < 15)
    def _():
        pltpu.sync_copy(mv0.at[pl.ds(0, RA - 24 * K)],
                        s_acc.at[pl.ds(base + 24 * K, RA - 24 * K)])

    @pl.when(sid == 15)
    def _():
        pltpu.sync_copy(mv0.at[pl.ds(0, RB - 24 * K)],
                        s_acc.at[pl.ds(base + 24 * K, RB - 24 * K)])

    plsc.subcore_barrier()

    def g_descs(jl, gv, av, gsem):
        return (pltpu.make_async_copy(hx_hbm.at[sloc.at[jl]], gv, gsem),
                pltpu.make_async_copy(ad_hbm.at[dloc.at[jl]], av, gsem))

    def start_gathers(jl, gv, av, gsem):
        d1, d2 = g_descs(jl, gv, av, gsem)
        d1.start()
        d2.start()

    def wait_gathers(jl, gv, av, gsem):
        d1, d2 = g_descs(jl, gv, av, gsem)
        d1.wait()
        d2.wait()

    def start_scatter(jl, mv, ssem):
        pltpu.make_async_copy(mv, s_acc.at[dloc.at[jl]], ssem).start(add=True)

    def wait_scatter(jl, mv, ssem):
        pltpu.make_async_copy(mv, s_acc.at[dloc.at[jl]], ssem).wait()

    def group(gv, av, mv, g, msk):
        rows = lane + (g * 16)
        asrc = plsc.load_gather(gv, [rows, c20])
        ad = plsc.load_gather(av, [rows, z16i])
        t = asrc + ad
        e = jnp.where(t >= 0.0, t, t * 0.2)
        w = jnp.exp(e)
        plsc.store_scatter(mv, [rows, c20], w, mask=msk)
        for c in range(HID):
            ci = jnp.full((16,), c, jnp.int32)
            h = plsc.load_gather(gv, [rows, ci])
            plsc.store_scatter(mv, [rows, ci], h * w, mask=msk)

    def compute_chunk(gv, av, mv):
        for g in range(K // 16):
            group(gv, av, mv, g, None)

    bufs = ((gv0, av0, mv0, gsem0, ssem0), (gv1, av1, mv1, gsem1, ssem1))

    HC = NCH // 2
    for half in range(2):
        pltpu.sync_copy(em_hbm.at[0, wid, pl.ds(HC * half, HC)], sloc)
        pltpu.sync_copy(em_hbm.at[1, wid, pl.ds(HC * half, HC)], dloc)

        start_gathers(0, gv0, av0, gsem0)
        start_gathers(1, gv1, av1, gsem1)

        def pair(i, carry):
            for b in range(2):
                gv, av, mv, gsem, ssem = bufs[b]
                jl = 2 * i + b
                wait_gathers(jl, gv, av, gsem)

                @pl.when(i > 0)
                def _():
                    wait_scatter(jl - 2, mv, ssem)

                compute_chunk(gv, av, mv)
                start_scatter(jl, mv, ssem)
                start_gathers(jl + 2, gv, av, gsem)
            return carry

        lax.fori_loop(0, HC // 2 - 1, pair, 0)

        wait_gathers(HC - 2, gv0, av0, gsem0)
        wait_scatter(HC - 4, mv0, ssem0)
        compute_chunk(gv0, av0, mv0)
        start_scatter(HC - 2, mv0, ssem0)

        wait_gathers(HC - 1, gv1, av1, gsem1)
        wait_scatter(HC - 3, mv1, ssem1)
        compute_chunk(gv1, av1, mv1)
        start_scatter(HC - 1, mv1, ssem1)

        wait_scatter(HC - 2, mv0, ssem0)
        wait_scatter(HC - 1, mv1, ssem1)

    plsc.subcore_barrier()

    @pl.when(sid < 15)
    def _():
        pltpu.sync_copy(s_acc.at[pl.ds(sid * RA, RA)],
                        out_hbm.at[cid, pl.ds(sid * RA, RA)])

    @pl.when(sid == 15)
    def _():
        pltpu.sync_copy(s_acc.at[pl.ds(15 * RA, RB)],
                        out_hbm.at[cid, pl.ds(15 * RA, RB)])


def _finish_body(p_ref, ow_ref, ob_ref, y_ref):
    s = p_ref[0] + p_ref[1]
    num = s[:, 0:20]
    den = s[:, 20:21]
    agg = num / (den + 1e-16) + ow_ref[0:1, :]
    h1 = jnp.maximum(agg, 0.0)
    y_ref[...] = (jnp.sum(h1 * ow_ref[1:2, :], axis=1, keepdims=True)
                  + ob_ref[0, 0])


def kernel(x, cnn_conv_w, cnn_conv_b, cnn_lin_w, cnn_lin_b,
           gat_lin_w, att_src, att_dst, gat_bias,
           out_lin_w, out_lin_b, edge_index):
    f32 = jnp.float32
    ind = np.zeros((395, PPAD, KERN), np.float32)
    for p in range(NPOS):
        for k in range(KERN):
            ind[3 + STRIDE * p + k, p, k] = 1.0
    w2 = cnn_conv_w[:, 0, :]
    wb = (ind.reshape(395 * PPAD, KERN) @ w2.T)
    wb = (wb.reshape(395, PPAD, CNN_CH).transpose(0, 2, 1)
          .reshape(395, CNN_CH * PPAD)).astype(jnp.bfloat16)
    pmask = (np.arange(PPAD) < NPOS).astype(np.float32)
    bvec = (cnn_conv_b[:, None] * pmask[None, :]).reshape(1, CNN_CH * PPAD)
    obd = np.zeros((CNN_CH * PPAD, CNN_CH), np.float32)
    for c in range(CNN_CH):
        obd[c * PPAD:c * PPAD + NPOS, c] = 1.0
    obd = jnp.asarray(obd)
    g80t = gat_lin_w[:, :80].T
    m1 = cnn_lin_w.T @ g80t
    c1 = cnn_lin_b @ g80t
    c1g = jnp.stack([gat_lin_w[:, 80], c1])
    attw = jnp.stack([att_src, att_dst], axis=1)
    ow = jnp.stack([gat_bias, out_lin_w[0]])
    ob = out_lin_b.reshape(1, 1)

    nblk = N // BN
    hx32, ad16 = pl.pallas_call(
        _dense_body,
        grid=(nblk,),
        in_specs=[
            pl.BlockSpec((BN, 395), lambda i: (i, 0)),
            pl.BlockSpec((395, CNN_CH * PPAD), lambda i: (0, 0)),
            pl.BlockSpec((1, CNN_CH * PPAD), lambda i: (0, 0)),
            pl.BlockSpec((CNN_CH * PPAD, CNN_CH), lambda i: (0, 0)),
            pl.BlockSpec((CNN_CH, HID), lambda i: (0, 0)),
            pl.BlockSpec((2, HID), lambda i: (0, 0)),
            pl.BlockSpec((HID, 2), lambda i: (0, 0)),
        ],
        out_specs=[
            pl.BlockSpec((BN, D), lambda i: (i, 0)),
            pl.BlockSpec((BN, 16), lambda i: (i, 0)),
        ],
        out_shape=[
            jax.ShapeDtypeStruct((N, D), f32),
            jax.ShapeDtypeStruct((N, 16), f32),
        ],
    )(x[:, 1, :], wb, bvec, obd, m1, c1g, attw)

    mesh = plsc.VectorSubcoreMesh(core_axis_name="c", subcore_axis_name="s",
                                  num_cores=2, num_subcores=16)
    pad = NW * NCH * K - E
    srcp = jnp.pad(edge_index[0], (0, pad))
    dstp = jnp.pad(edge_index[1], (0, pad), constant_values=N)
    em = jnp.stack([srcp, dstp]).reshape(2, NW, NCH, K)
    adp = jnp.pad(ad16, ((0, 8), (0, 0)))
    edge_call = functools.partial(
        pl.kernel,
        out_type=jax.ShapeDtypeStruct((2, N, D), f32),
        mesh=mesh,
        compiler_params=pltpu.CompilerParams(needs_layout_passes=False,
                                             use_tc_tiling_on_sc=False),
        scratch_types=[
            pltpu.VMEM((NCH // 2, K), jnp.int32),
            pltpu.VMEM((NCH // 2, K), jnp.int32),
            pltpu.VMEM((K, D), f32),
            pltpu.VMEM((K, 16), f32),
            pltpu.VMEM((K, D), f32),
            pltpu.VMEM((K, D), f32),
            pltpu.VMEM((K, 16), f32),
            pltpu.VMEM((K, D), f32),
            pltpu.VMEM_SHARED((N + 8, D), f32),
            pltpu.SemaphoreType.DMA,
            pltpu.SemaphoreType.DMA,
            pltpu.SemaphoreType.DMA,
            pltpu.SemaphoreType.DMA,
        ],
    )(_edge_body)
    partials = edge_call(em, hx32, adp)

    nblk2 = N // BN
    y = pl.pallas_call(
        _finish_body,
        grid=(nblk2,),
        in_specs=[
            pl.BlockSpec((2, BN, D), lambda i: (0, i, 0)),
            pl.BlockSpec((2, HID), lambda i: (0, 0)),
            pl.BlockSpec(memory_space=pltpu.SMEM),
        ],
        out_specs=pl.BlockSpec((BN, 1), lambda i: (i, 0)),
        out_shape=jax.ShapeDtypeStruct((N, 1), f32),
    )(partials, ow, ob)
    return y
